# baseline (speedup 1.0000x reference)
import functools

import jax
import jax.numpy as jnp
from jax import lax
from jax.experimental import pallas as pl
from jax.experimental.pallas import tpu as pltpu
from jax.experimental.pallas import tpu_sc as plsc

ROWS = 128
COLS = 32768
LANES = 16
NUM_CORES = 2
NUM_SUBCORES = 16
NUM_WORKERS = NUM_CORES * NUM_SUBCORES
ROWS_PER_WORKER = ROWS // NUM_WORKERS
CHUNK = 4096
CHUNKS_PER_ROW = COLS // CHUNK
UNROLL = 4
ELEMS_PER_STEP = UNROLL * LANES
STEPS_PER_CHUNK = CHUNK // ELEMS_PER_STEP
STEPS_PER_ROW = STEPS_PER_CHUNK * CHUNKS_PER_ROW

_INT_MAX = 2147483647


def _argmin_body(x_hbm, out_hbm, buf, out_v, sem0, sem1):
    sems = (sem0, sem1)
    wid = lax.axis_index("s") * NUM_CORES + lax.axis_index("c")
    row0 = wid * ROWS_PER_WORKER
    iota = lax.iota(jnp.int32, LANES)
    total_chunks = ROWS_PER_WORKER * CHUNKS_PER_ROW

    def start(t):
        r, c = divmod(t, CHUNKS_PER_ROW)
        b = t & 1
        return pltpu.async_copy(
            x_hbm.at[row0 + r, pl.ds(c * CHUNK, CHUNK)], buf.at[b], sems[b])

    copies = [None, None]
    copies[0] = start(0)
    result_v = jnp.zeros((LANES,), jnp.int32)

    for r in range(ROWS_PER_WORKER):
        vals = [jnp.full((LANES,), jnp.inf, jnp.float32) for _ in range(UNROLL)]
        stamps = [jnp.zeros((LANES,), jnp.int32) for _ in range(UNROLL)]

        for c in range(CHUNKS_PER_ROW):
            t = r * CHUNKS_PER_ROW + c
            b = t & 1
            if t + 1 < total_chunks:
                copies[(t + 1) & 1] = start(t + 1)
            copies[b].wait()

            def chunk_body(i, carry, b=b, c=c):
                v0, v1, v2, v3, s0, s1, s2, s3 = carry
                vs = [v0, v1, v2, v3]
                ss = [s0, s1, s2, s3]
                step = c * STEPS_PER_CHUNK + i
                step_v = jnp.full((LANES,), 0, jnp.int32) + step
                off = i * ELEMS_PER_STEP
                for u in range(UNROLL):
                    v = buf[b, pl.ds(off + u * LANES, LANES)]
                    m = v < vs[u]
                    vs[u] = jnp.where(m, v, vs[u])
                    ss[u] = jnp.where(m, step_v, ss[u])
                return (*vs, *ss)

            carry = (*vals, *stamps)
            carry = lax.fori_loop(0, STEPS_PER_CHUNK, chunk_body, carry)
            vals = list(carry[:UNROLL])
            stamps = list(carry[UNROLL:])

        idxs = [stamps[u] * ELEMS_PER_STEP + (u * LANES) + iota
                for u in range(UNROLL)]
        pairs = list(zip(vals, idxs))
        while len(pairs) > 1:
            nxt = []
            for j in range(0, len(pairs), 2):
                (av, ai), (bv, bi) = pairs[j], pairs[j + 1]
                take_b = (bv < av) | ((bv == av) & (bi < ai))
                nxt.append((jnp.where(take_b, bv, av),
                            jnp.where(take_b, bi, ai)))
            pairs = nxt
        mv, mi = pairs[0]

        rowmin = jnp.min(mv)
        cand = jnp.where(mv == rowmin, mi, jnp.int32(_INT_MAX))
        rowarg = jnp.min(cand)
        result_v = jnp.where(iota == r, rowarg, result_v)

    out_v[...] = result_v
    pltpu.sync_copy(out_v, out_hbm.at[wid])


def kernel(x):
    mesh = plsc.VectorSubcoreMesh(core_axis_name="c", subcore_axis_name="s")
    k = functools.partial(
        pl.kernel,
        mesh=mesh,
        out_type=jax.ShapeDtypeStruct((NUM_WORKERS, LANES), jnp.int32),
        scratch_types=[
            pltpu.VMEM((2, CHUNK), jnp.float32),
            pltpu.VMEM((LANES,), jnp.int32),
            pltpu.SemaphoreType.DMA,
            pltpu.SemaphoreType.DMA,
        ],
        compiler_params=pltpu.CompilerParams(needs_layout_passes=False),
    )(_argmin_body)
    y = k(x)
    return y[:, :ROWS_PER_WORKER].reshape(ROWS, 1)


# speedup vs baseline: 1.1280x; 1.1280x over previous
import functools

import jax
import jax.numpy as jnp
from jax import lax
from jax.experimental import pallas as pl
from jax.experimental.pallas import tpu as pltpu
from jax.experimental.pallas import tpu_sc as plsc

ROWS = 128
COLS = 32768
LANES = 16
NUM_CORES = 2
NUM_SUBCORES = 16
NUM_WORKERS = NUM_CORES * NUM_SUBCORES
ROWS_PER_WORKER = ROWS // NUM_WORKERS
CHUNK = 4096
CHUNKS_PER_ROW = COLS // CHUNK
NACC = 8
P1_ELEMS = NACC * LANES
P1_STEPS = CHUNK // P1_ELEMS
NACC2 = 4
P2_ELEMS = NACC2 * LANES
P2_STEPS = CHUNK // P2_ELEMS

_INT_MAX = 2147483647


def _argmin_body(x_hbm, out_hbm, buf, out_v, sem0, sem1):
    sems = (sem0, sem1)
    wid = lax.axis_index("s") * NUM_CORES + lax.axis_index("c")
    row0 = wid * ROWS_PER_WORKER
    iota = lax.iota(jnp.int32, LANES)

    def start(r):
        return pltpu.async_copy(x_hbm.at[row0 + r], buf.at[r & 1], sems[r & 1])

    copies = [None, None]
    copies[0] = start(0)
    result_v = jnp.zeros((LANES,), jnp.int32)

    for r in range(ROWS_PER_WORKER):
        b = r & 1
        if r + 1 < ROWS_PER_WORKER:
            copies[(r + 1) & 1] = start(r + 1)
        copies[b].wait()

        chunkvecs = []
        for c in range(CHUNKS_PER_ROW):
            accs = tuple(jnp.full((LANES,), jnp.inf, jnp.float32)
                         for _ in range(NACC))

            def p1_body(i, carry, b=b, c=c):
                vs = list(carry)
                off = c * CHUNK + i * P1_ELEMS
                for u in range(NACC):
                    v = buf[b, pl.ds(off + u * LANES, LANES)]
                    vs[u] = jnp.minimum(vs[u], v)
                return tuple(vs)

            accs = plsc.parallel_loop(0, P1_STEPS, 1, carry=accs)(p1_body)
            accs = list(accs)
            while len(accs) > 1:
                accs = [jnp.minimum(accs[j], accs[j + 1])
                        for j in range(0, len(accs), 2)]
            chunkvecs.append(accs[0])

        mv = chunkvecs[0]
        for c in range(1, CHUNKS_PER_ROW):
            mv = jnp.minimum(mv, chunkvecs[c])
        rowmin = jnp.min(mv)

        chunk_id = jnp.int32(0)
        for c in reversed(range(CHUNKS_PER_ROW)):
            has = jnp.any(chunkvecs[c] == rowmin)
            chunk_id = jnp.where(has, jnp.int32(c), chunk_id)

        base = chunk_id * CHUNK
        minis = tuple(jnp.full((LANES,), _INT_MAX, jnp.int32)
                      for _ in range(NACC2))

        def p2_body(i, carry, b=b):
            ms = list(carry)
            off = base + i * P2_ELEMS
            for u in range(NACC2):
                v = buf[b, pl.ds(off + u * LANES, LANES)]
                idxv = iota + (off + u * LANES)
                cand = jnp.where(v == rowmin, idxv, jnp.int32(_INT_MAX))
                ms[u] = jnp.minimum(ms[u], cand)
            return tuple(ms)

        minis = plsc.parallel_loop(0, P2_STEPS, 1, carry=minis)(p2_body)
        minis = list(minis)
        while len(minis) > 1:
            minis = [jnp.minimum(minis[j], minis[j + 1])
                     for j in range(0, len(minis), 2)]
        rowarg = jnp.min(minis[0])
        result_v = jnp.where(iota == r, rowarg, result_v)

    out_v[...] = result_v
    pltpu.sync_copy(out_v, out_hbm.at[wid])


def kernel(x):
    mesh = plsc.VectorSubcoreMesh(core_axis_name="c", subcore_axis_name="s")
    k = functools.partial(
        pl.kernel,
        mesh=mesh,
        out_type=jax.ShapeDtypeStruct((NUM_WORKERS, LANES), jnp.int32),
        scratch_types=[
            pltpu.VMEM((2, COLS), jnp.float32),
            pltpu.VMEM((LANES,), jnp.int32),
            pltpu.SemaphoreType.DMA,
            pltpu.SemaphoreType.DMA,
        ],
        compiler_params=pltpu.CompilerParams(needs_layout_passes=False),
    )(_argmin_body)
    y = k(x)
    return y[:, :ROWS_PER_WORKER].reshape(ROWS, 1)
